# Initial kernel scaffold; baseline (speedup 1.0000x reference)
import jax
import jax.numpy as jnp
from jax.experimental import pallas as pl


def kernel(atom_indices, subshell_embeds, atom_configs):
    raise NotImplementedError("write your pallas kernel here")


# speedup vs baseline: 1.6549x; 1.6549x over previous
import functools

import jax
import jax.numpy as jnp
from jax import lax
from jax.experimental import pallas as pl
from jax.experimental.pallas import tpu as pltpu
from jax.experimental.pallas import tpu_sc as plsc

N_ROWS_TABLE = 25
N_SUB = 12
D = 128
CHUNK = 128


def _table_body(cfg_ref, emb_ref, out_ref):
    out_ref[...] = lax.dot_general(
        cfg_ref[...], emb_ref[...],
        dimension_numbers=(((1,), (0,)), ((), ())),
        preferred_element_type=jnp.float32,
    )


def _fuse_table(atom_configs, subshell_embeds):
    return pl.pallas_call(
        _table_body,
        out_shape=jax.ShapeDtypeStruct((N_ROWS_TABLE, D), jnp.float32),
    )(atom_configs, subshell_embeds)


def _make_gather(n_rows):
    info = plsc.get_sparse_core_info()
    nc, ns = info.num_cores, info.num_subcores
    nw = nc * ns
    assert n_rows % (nw * CHUNK) == 0
    rows_per_w = n_rows // nw
    nch = rows_per_w // CHUNK
    mesh = plsc.VectorSubcoreMesh(core_axis_name="c", subcore_axis_name="s")

    @functools.partial(
        pl.kernel,
        mesh=mesh,
        out_type=jax.ShapeDtypeStruct((n_rows, D), jnp.float32),
        scratch_types=[
            pltpu.VMEM((rows_per_w,), jnp.int32),
            pltpu.VMEM((CHUNK, D), jnp.float32),
            pltpu.VMEM((CHUNK, D), jnp.float32),
            pltpu.SemaphoreType.DMA,
            pltpu.SemaphoreType.DMA,
        ],
    )
    def gather(table_hbm, idx_hbm, out_hbm, idx_v, buf0, buf1, sem0, sem1):
        wid = lax.axis_index("s") * nc + lax.axis_index("c")
        base = wid * rows_per_w
        bufs = (buf0, buf1)
        sems = (sem0, sem1)

        pltpu.sync_copy(idx_hbm.at[pl.ds(base, rows_per_w)], idx_v)

        def start_gather(jj, b):
            pltpu.async_copy(
                table_hbm.at[idx_v.at[pl.ds(jj * CHUNK, CHUNK)]], bufs[b], sems[b])

        def finish_chunk(jj, b):
            pltpu.make_async_copy(
                table_hbm.at[idx_v.at[pl.ds(jj * CHUNK, CHUNK)]], bufs[b], sems[b]).wait()
            pltpu.sync_copy(bufs[b], out_hbm.at[pl.ds(base + jj * CHUNK, CHUNK)])

        start_gather(0, 0)
        start_gather(1, 1)

        def body(i, carry):
            for b in range(2):
                jj = 2 * i + b
                finish_chunk(jj, b)
                start_gather(jj + 2, b)
            return carry

        lax.fori_loop(0, nch // 2 - 1, body, 0)
        finish_chunk(nch - 2, 0)
        finish_chunk(nch - 1, 1)

    return gather


def kernel(atom_indices, subshell_embeds, atom_configs):
    b, h = atom_indices.shape
    n_rows = b * h
    table = _fuse_table(atom_configs, subshell_embeds)
    idx_flat = atom_indices.astype(jnp.int32).reshape(n_rows)
    out = _make_gather(n_rows)(table, idx_flat)
    return out.reshape(b, h, D)


# speedup vs baseline: 8.5398x; 5.1603x over previous
import functools

import jax
import jax.numpy as jnp
from jax import lax
from jax.experimental import pallas as pl
from jax.experimental.pallas import tpu as pltpu
from jax.experimental.pallas import tpu_sc as plsc

N_CFG = 25
N_CFG_PAD = 32
N_SUB = 12
D = 128
L = 16
G = 4


def _splat(val, n=L):
    return jnp.full((n,), val, jnp.int32)


def _make_sc_kernel(batch, hist):
    info = plsc.get_sparse_core_info()
    nc, ns = info.num_cores, info.num_subcores
    nw = nc * ns
    assert batch % (nw * G) == 0
    items_per_w = batch // nw
    nch = items_per_w // G
    mesh = plsc.VectorSubcoreMesh(core_axis_name="c", subcore_axis_name="s")

    @functools.partial(
        pl.kernel,
        mesh=mesh,
        out_type=jax.ShapeDtypeStruct((batch, hist, D), jnp.float32),
        scratch_types=[
            pltpu.VMEM((N_CFG, D), jnp.float32),
            pltpu.VMEM((N_SUB, D), jnp.float32),
            pltpu.VMEM((2, D), jnp.float32),
            pltpu.VMEM((items_per_w, hist), jnp.int32),
            pltpu.VMEM((G, hist, D), jnp.float32),
            pltpu.VMEM((G, hist, D), jnp.float32),
            pltpu.VMEM_SHARED((N_CFG_PAD, D), jnp.float32),
            pltpu.SemaphoreType.DMA,
            pltpu.SemaphoreType.DMA,
        ],
    )
    def sc_kernel(idx_hbm, emb_hbm, cfg_hbm, out_hbm,
                  cfg_v, emb_v, row_v, idx_v, buf0, buf1, table_sh,
                  sem0, sem1):
        sid = lax.axis_index("s")
        cid = lax.axis_index("c")
        wid = sid * nc + cid
        bufs = (buf0, buf1)
        sems = (sem0, sem1)

        pltpu.sync_copy(cfg_hbm, cfg_v)
        pltpu.sync_copy(emb_hbm, emb_v)
        r0 = 2 * sid
        for b in range(2):
            r = jnp.minimum(r0 + b, N_CFG - 1)
            cfg_row = cfg_v[r, pl.ds(0, L)]
            dn = lax.GatherDimensionNumbers(
                offset_dims=(), collapsed_slice_dims=(0,), start_index_map=(0,))
            cs = [lax.gather(cfg_row, _splat(s)[:, None], dn, (1,),
                             mode=lax.GatherScatterMode.PROMISE_IN_BOUNDS)
                  for s in range(N_SUB)]
            for g in range(D // L):
                acc = jnp.zeros((L,), jnp.float32)
                for s in range(N_SUB):
                    acc = acc + cs[s] * emb_v[s, pl.ds(g * L, L)]
                row_v[b, pl.ds(g * L, L)] = acc
        pltpu.sync_copy(row_v, table_sh.at[pl.ds(r0, 2)])
        plsc.subcore_barrier()

        base = wid * items_per_w
        pltpu.sync_copy(idx_hbm.at[pl.ds(base, items_per_w)], idx_v)

        def issue(cc, b):
            for g in range(G):
                pltpu.async_copy(
                    table_sh.at[idx_v.at[cc * G + g]], bufs[b].at[g], sems[b])

        def drain_write(cc, b):
            pltpu.make_async_copy(
                out_hbm.at[pl.ds(0, G)], bufs[b], sems[b]).wait()
            pltpu.sync_copy(bufs[b], out_hbm.at[pl.ds(base + cc * G, G)])

        issue(0, 0)
        issue(1, 1)

        def body(i, carry):
            for b in range(2):
                cc = 2 * i + b
                drain_write(cc, b)
                issue(cc + 2, b)
            return carry

        lax.fori_loop(0, nch // 2 - 1, body, 0)
        drain_write(nch - 2, 0)
        drain_write(nch - 1, 1)

    return sc_kernel


def kernel(atom_indices, subshell_embeds, atom_configs):
    batch, hist = atom_indices.shape
    idx = atom_indices.astype(jnp.int32)
    cfg = jnp.pad(atom_configs, ((0, 0), (0, D - atom_configs.shape[1])))
    return _make_sc_kernel(batch, hist)(idx, subshell_embeds, cfg)


# speedup vs baseline: 18.2716x; 2.1396x over previous
import functools

import jax
import jax.numpy as jnp
from jax import lax
from jax.experimental import pallas as pl
from jax.experimental.pallas import tpu as pltpu
from jax.experimental.pallas import tpu_sc as plsc

N_CFG = 25
N_CFG_PAD = 32
N_SUB = 12
D = 128
L = 16
G = 4


def _splat(val, n=L):
    return jnp.full((n,), val, jnp.int32)


def _make_sc_kernel(batch, hist):
    info = plsc.get_sparse_core_info()
    nc, ns = info.num_cores, info.num_subcores
    nw = nc * ns
    assert batch % nw == 0
    items_per_w = batch // nw
    mesh = plsc.VectorSubcoreMesh(core_axis_name="c", subcore_axis_name="s")

    @functools.partial(
        pl.kernel,
        mesh=mesh,
        out_type=jax.ShapeDtypeStruct((hist * batch, D), jnp.float32),
        scratch_types=[
            pltpu.VMEM((N_CFG, D), jnp.float32),
            pltpu.VMEM((N_SUB, D), jnp.float32),
            pltpu.VMEM((2, D), jnp.float32),
            pltpu.VMEM((hist, items_per_w), jnp.int32),
            pltpu.VMEM((items_per_w, D), jnp.float32),
            pltpu.VMEM((items_per_w, D), jnp.float32),
            pltpu.VMEM_SHARED((N_CFG_PAD, D), jnp.float32),
            pltpu.SemaphoreType.DMA,
            pltpu.SemaphoreType.DMA,
        ],
    )
    def sc_kernel(idx_hbm, emb_hbm, cfg_hbm, out_hbm,
                  cfg_v, emb_v, row_v, idx_v, buf0, buf1, table_sh,
                  sem0, sem1):
        sid = lax.axis_index("s")
        cid = lax.axis_index("c")
        wid = sid * nc + cid
        bufs = (buf0, buf1)
        sems = (sem0, sem1)

        pltpu.sync_copy(cfg_hbm, cfg_v)
        pltpu.sync_copy(emb_hbm, emb_v)
        r0 = 2 * sid
        for b in range(2):
            r = jnp.minimum(r0 + b, N_CFG - 1)
            cfg_row = cfg_v[r, pl.ds(0, L)]
            dn = lax.GatherDimensionNumbers(
                offset_dims=(), collapsed_slice_dims=(0,), start_index_map=(0,))
            cs = [lax.gather(cfg_row, _splat(s)[:, None], dn, (1,),
                             mode=lax.GatherScatterMode.PROMISE_IN_BOUNDS)
                  for s in range(N_SUB)]
            for g in range(D // L):
                acc = jnp.zeros((L,), jnp.float32)
                for s in range(N_SUB):
                    acc = acc + cs[s] * emb_v[s, pl.ds(g * L, L)]
                row_v[b, pl.ds(g * L, L)] = acc
        pltpu.sync_copy(row_v, table_sh.at[pl.ds(r0, 2)])
        plsc.subcore_barrier()

        b0 = wid * items_per_w
        pltpu.sync_copy(
            idx_hbm.at[pl.ds(0, hist), pl.ds(b0, items_per_w)], idx_v)

        def issue(l, b):
            pltpu.async_copy(table_sh.at[idx_v.at[l]], bufs[b], sems[b])

        def drain_write(l, b):
            pltpu.make_async_copy(
                table_sh.at[idx_v.at[l]], bufs[b], sems[b]).wait()
            pltpu.sync_copy(
                bufs[b], out_hbm.at[pl.ds(l * batch + b0, items_per_w)])

        issue(0, 0)
        issue(1, 1)

        def body(i, carry):
            for b in range(2):
                l = 2 * i + b
                drain_write(l, b)
                issue(l + 2, b)
            return carry

        lax.fori_loop(0, hist // 2 - 1, body, 0)
        drain_write(hist - 2, 0)
        drain_write(hist - 1, 1)

    return sc_kernel


def kernel(atom_indices, subshell_embeds, atom_configs):
    batch, hist = atom_indices.shape
    idx_t = atom_indices.astype(jnp.int32).T
    cfg = jnp.pad(atom_configs, ((0, 0), (0, D - atom_configs.shape[1])))
    out = _make_sc_kernel(batch, hist)(idx_t, subshell_embeds, cfg)
    return out.reshape(hist, batch, D).transpose(1, 0, 2)


# speedup vs baseline: 18.4393x; 1.0092x over previous
import functools

import jax
import jax.numpy as jnp
from jax import lax
from jax.experimental import pallas as pl
from jax.experimental.pallas import tpu as pltpu
from jax.experimental.pallas import tpu_sc as plsc

N_CFG = 25
N_CFG_PAD = 32
N_SUB = 12
D = 128
L = 16
G = 4


def _splat(val, n=L):
    return jnp.full((n,), val, jnp.int32)


def _make_sc_kernel(batch, hist):
    info = plsc.get_sparse_core_info()
    nc, ns = info.num_cores, info.num_subcores
    nw = nc * ns
    assert batch % nw == 0
    items_per_w = batch // nw
    mesh = plsc.VectorSubcoreMesh(core_axis_name="c", subcore_axis_name="s")

    @functools.partial(
        pl.kernel,
        mesh=mesh,
        out_type=jax.ShapeDtypeStruct((hist * batch, D), jnp.float32),
        scratch_types=[
            pltpu.VMEM((N_CFG, D), jnp.float32),
            pltpu.VMEM((N_SUB, D), jnp.float32),
            pltpu.VMEM((2, D), jnp.float32),
            pltpu.VMEM((hist, items_per_w), jnp.int32),
            pltpu.VMEM((items_per_w, D), jnp.float32),
            pltpu.VMEM((items_per_w, D), jnp.float32),
            pltpu.VMEM((items_per_w, D), jnp.float32),
            pltpu.VMEM((items_per_w, D), jnp.float32),
            pltpu.VMEM_SHARED((N_CFG_PAD, D), jnp.float32),
            pltpu.SemaphoreType.DMA,
            pltpu.SemaphoreType.DMA,
            pltpu.SemaphoreType.DMA,
            pltpu.SemaphoreType.DMA,
            pltpu.SemaphoreType.DMA,
            pltpu.SemaphoreType.DMA,
            pltpu.SemaphoreType.DMA,
            pltpu.SemaphoreType.DMA,
            pltpu.SemaphoreType.DMA,
        ],
    )
    def sc_kernel(idx_hbm, emb_hbm, cfg_hbm, out_hbm,
                  cfg_v, emb_v, row_v, idx_v, buf0, buf1, buf2, buf3, table_sh,
                  gsem0, gsem1, gsem2, gsem3,
                  wsem0, wsem1, wsem2, wsem3, isem):
        sid = lax.axis_index("s")
        cid = lax.axis_index("c")
        wid = sid * nc + cid
        bufs = (buf0, buf1, buf2, buf3)
        gsems = (gsem0, gsem1, gsem2, gsem3)
        wsems = (wsem0, wsem1, wsem2, wsem3)

        b0 = wid * items_per_w
        pltpu.async_copy(
            idx_hbm.at[pl.ds(0, hist), pl.ds(b0, items_per_w)], idx_v, isem)

        pltpu.sync_copy(cfg_hbm, cfg_v)
        pltpu.sync_copy(emb_hbm, emb_v)
        r0 = 2 * sid
        for b in range(2):
            r = jnp.minimum(r0 + b, N_CFG - 1)
            cfg_row = cfg_v[r, pl.ds(0, L)]
            dn = lax.GatherDimensionNumbers(
                offset_dims=(), collapsed_slice_dims=(0,), start_index_map=(0,))
            cs = [lax.gather(cfg_row, _splat(s)[:, None], dn, (1,),
                             mode=lax.GatherScatterMode.PROMISE_IN_BOUNDS)
                  for s in range(N_SUB)]
            for g in range(D // L):
                acc = jnp.zeros((L,), jnp.float32)
                for s in range(N_SUB):
                    acc = acc + cs[s] * emb_v[s, pl.ds(g * L, L)]
                row_v[b, pl.ds(g * L, L)] = acc
        pltpu.sync_copy(row_v, table_sh.at[pl.ds(r0, 2)])
        plsc.subcore_barrier()

        pltpu.make_async_copy(
            idx_hbm.at[pl.ds(0, hist), pl.ds(b0, items_per_w)], idx_v,
            isem).wait()

        def issue_gather(l, s):
            pltpu.async_copy(table_sh.at[idx_v.at[l]], bufs[s], gsems[s])

        def wait_gather(s):
            pltpu.make_async_copy(
                table_sh.at[idx_v.at[0]], bufs[s], gsems[s]).wait()

        def issue_write(l, s):
            pltpu.async_copy(
                bufs[s], out_hbm.at[pl.ds(l * batch + b0, items_per_w)],
                wsems[s])

        def wait_write(s):
            pltpu.make_async_copy(
                bufs[s], out_hbm.at[pl.ds(b0, items_per_w)], wsems[s]).wait()

        issue_gather(0, 0)
        issue_gather(1, 1)
        for l in (0, 1):
            wait_gather(l)
            issue_write(l, l)
            issue_gather(l + 2, l + 2)

        def body(i, carry):
            for k in range(4):
                l = 4 * i + 2 + k
                s = (2 + k) % 4
                wait_gather(s)
                issue_write(l, s)
                wait_write(k)
                issue_gather(l + 2, k)
            return carry

        lax.fori_loop(0, (hist - 6) // 4, body, 0)
        for k in range(2):
            l = hist - 4 + k
            s = l % 4
            wait_gather(s)
            issue_write(l, s)
            wait_write(k)
            issue_gather(l + 2, k)
        for k in range(2):
            s = (hist - 2 + k) % 4
            wait_gather(s)
            issue_write(hist - 2 + k, s)
        for s in range(4):
            wait_write(s)

    return sc_kernel


def kernel(atom_indices, subshell_embeds, atom_configs):
    batch, hist = atom_indices.shape
    idx_t = atom_indices.astype(jnp.int32).T
    cfg = jnp.pad(atom_configs, ((0, 0), (0, D - atom_configs.shape[1])))
    out = _make_sc_kernel(batch, hist)(idx_t, subshell_embeds, cfg)
    return out.reshape(hist, batch, D).transpose(1, 0, 2)
